# baseline (speedup 1.0000x reference)
import functools

import jax
import jax.numpy as jnp
from jax import lax
from jax.experimental import pallas as pl
from jax.experimental.pallas import tpu as pltpu
from jax.experimental.pallas import tpu_sc as plsc

N = 10000
NPAD = 10112
E = 320000
D_IN = 128
D_HID = 128
D_OUT = 16
NC, NS = 2, 16
NW = NC * NS
CH = 128
EPT = E // NW
NCH = 80
EPT_PAD = NCH * CH
RPT = NPAD // NS
R_TC = 2528

_mesh = plsc.VectorSubcoreMesh(
    core_axis_name="c", subcore_axis_name="s", num_cores=NC, num_subcores=NS)


@functools.partial(
    pl.kernel,
    out_type=jax.ShapeDtypeStruct((NC, 2, NPAD, 8), jnp.float32),
    mesh=_mesh,
    compiler_params=pltpu.CompilerParams(use_tc_tiling_on_sc=False),
    scratch_types=[
        pltpu.VMEM((NCH, CH), jnp.int32),
        pltpu.VMEM((NCH, CH), jnp.int32),
        pltpu.VMEM((CH, 8), jnp.float32),
        pltpu.VMEM_SHARED((NPAD, 8), jnp.float32),
        pltpu.VMEM_SHARED((NPAD, 8), jnp.float32),
    ],
)
def _degrees(src_hbm, dst_hbm, ones_hbm, zeros_hbm, out_hbm,
             src_v, dst_v, ones_v, acc_s, acc_d):
    cid = lax.axis_index("c")
    sid = lax.axis_index("s")
    wid = cid * NS + sid
    pltpu.sync_copy(zeros_hbm, acc_s.at[pl.ds(sid * RPT, RPT)])
    pltpu.sync_copy(zeros_hbm, acc_d.at[pl.ds(sid * RPT, RPT)])
    pltpu.sync_copy(ones_hbm, ones_v)
    pltpu.sync_copy(src_hbm.at[wid], src_v)
    pltpu.sync_copy(dst_hbm.at[wid], dst_v)
    plsc.subcore_barrier()

    def body(j, carry):
        pltpu.sync_copy(ones_v, acc_s.at[src_v.at[j]], add=True)
        pltpu.sync_copy(ones_v, acc_d.at[dst_v.at[j]], add=True)
        return carry

    lax.fori_loop(0, NCH, body, 0)
    plsc.subcore_barrier()
    rows = pl.ds(sid * RPT, RPT)
    pltpu.sync_copy(acc_s.at[rows], out_hbm.at[cid, 0, rows])
    pltpu.sync_copy(acc_d.at[rows], out_hbm.at[cid, 1, rows])


def _make_segsum(D, ch, qi, pipelined):
    nch = EPT_PAD // ch
    nq = nch // qi

    @functools.partial(
        pl.kernel,
        out_type=jax.ShapeDtypeStruct((NC, NPAD, D), jnp.float32),
        mesh=_mesh,
        compiler_params=pltpu.CompilerParams(use_tc_tiling_on_sc=(D == 128)),
        scratch_types=(
            [pltpu.VMEM((nq, ch), jnp.int32),
             pltpu.VMEM((nq, ch), jnp.int32)]
            + [pltpu.VMEM((ch, D), jnp.float32)] * (2 if pipelined else 1)
            + [pltpu.VMEM_SHARED((NPAD, D), jnp.float32)]
            + [pltpu.SemaphoreType.DMA] * (2 if pipelined else 1)
        ),
    )
    def segsum(table_hbm, src_hbm, dst_hbm, zeros_hbm, out_hbm,
               src_v, dst_v, *rest):
        if pipelined:
            rows0, rows1, acc, sem0, sem1 = rest
        else:
            rows0, acc, sem0 = rest
        cid = lax.axis_index("c")
        sid = lax.axis_index("s")
        wid = cid * NS + sid
        pltpu.sync_copy(zeros_hbm, acc.at[pl.ds(sid * RPT, RPT)])
        plsc.subcore_barrier()

        for q in range(qi):
            pltpu.sync_copy(src_hbm.at[wid, pl.ds(q * nq, nq)], src_v)
            pltpu.sync_copy(dst_hbm.at[wid, pl.ds(q * nq, nq)], dst_v)
            if pipelined:
                pltpu.async_copy(table_hbm.at[src_v.at[0]], rows0, sem0)
                pltpu.async_copy(table_hbm.at[src_v.at[1]], rows1, sem1)

                @pl.loop(0, nq - 2, step=2)
                def _(j):
                    pltpu.make_async_copy(table_hbm.at[src_v.at[j]], rows0, sem0).wait()
                    pltpu.sync_copy(rows0, acc.at[dst_v.at[j]], add=True)
                    pltpu.async_copy(table_hbm.at[src_v.at[j + 2]], rows0, sem0)
                    pltpu.make_async_copy(table_hbm.at[src_v.at[j + 1]], rows1, sem1).wait()
                    pltpu.sync_copy(rows1, acc.at[dst_v.at[j + 1]], add=True)
                    pltpu.async_copy(table_hbm.at[src_v.at[j + 3]], rows1, sem1)

                pltpu.make_async_copy(table_hbm.at[src_v.at[nq - 2]], rows0, sem0).wait()
                pltpu.sync_copy(rows0, acc.at[dst_v.at[nq - 2]], add=True)
                pltpu.make_async_copy(table_hbm.at[src_v.at[nq - 1]], rows1, sem1).wait()
                pltpu.sync_copy(rows1, acc.at[dst_v.at[nq - 1]], add=True)
            else:
                @pl.loop(0, nq)
                def _(j):
                    pltpu.async_copy(table_hbm.at[src_v.at[j]], rows0, sem0).wait()
                    pltpu.sync_copy(rows0, acc.at[dst_v.at[j]], add=True)
        plsc.subcore_barrier()
        rows = pl.ds(sid * RPT, RPT)
        pltpu.sync_copy(acc.at[rows], out_hbm.at[cid, rows])

    return segsum


CH128 = 128
_segsum128 = _make_segsum(D_HID, CH128, 1, pipelined=False)
_segsum16 = _make_segsum(D_OUT, CH, 1, pipelined=True)


def _norm_from(deg_ref, which):
    deg = deg_ref[0, which][:, :1] + deg_ref[1, which][:, :1]
    return lax.rsqrt(jnp.maximum(deg, 1.0))


def _mm1_body(x_ref, w_ref, deg_ref, o_ref):
    norm_out = _norm_from(deg_ref, 0)
    o_ref[...] = jnp.dot(x_ref[...], w_ref[...],
                         preferred_element_type=jnp.float32) * norm_out


def _mm2_body(agg_ref, deg_ref, b1_ref, w2_ref, o_ref):
    agg = agg_ref[0] + agg_ref[1]
    norm_in = _norm_from(deg_ref, 1)
    norm_out = _norm_from(deg_ref, 0)
    h = jnp.maximum(agg * norm_in + b1_ref[...], 0.0)
    h2 = jnp.dot(h, w2_ref[...], preferred_element_type=jnp.float32) * norm_out
    rows = lax.broadcasted_iota(jnp.int32, (R_TC, 1), 0) + pl.program_id(0) * R_TC
    o_ref[...] = jnp.where(rows < N, h2, 0.0)


def _final_body(agg_ref, deg_ref, b2_ref, o_ref):
    agg = agg_ref[0] + agg_ref[1]
    norm_in = _norm_from(deg_ref, 1)
    o_ref[...] = agg * norm_in + b2_ref[...]


_DEG_SPEC = pl.BlockSpec((NC, 2, R_TC, 8), lambda i: (0, 0, i, 0))


def _mm1(x, w1, degs):
    return pl.pallas_call(
        _mm1_body,
        grid=(NPAD // R_TC,),
        in_specs=[
            pl.BlockSpec((R_TC, D_IN), lambda i: (i, 0)),
            pl.BlockSpec((D_IN, D_HID), lambda i: (0, 0)),
            _DEG_SPEC,
        ],
        out_specs=pl.BlockSpec((R_TC, D_HID), lambda i: (i, 0)),
        out_shape=jax.ShapeDtypeStruct((NPAD, D_HID), jnp.float32),
    )(x, w1, degs)


def _mm2(agg, degs, b1, w2):
    return pl.pallas_call(
        _mm2_body,
        grid=(NPAD // R_TC,),
        in_specs=[
            pl.BlockSpec((NC, R_TC, D_HID), lambda i: (0, i, 0)),
            _DEG_SPEC,
            pl.BlockSpec((1, D_HID), lambda i: (0, 0)),
            pl.BlockSpec((D_HID, D_OUT), lambda i: (0, 0)),
        ],
        out_specs=pl.BlockSpec((R_TC, D_OUT), lambda i: (i, 0)),
        out_shape=jax.ShapeDtypeStruct((NPAD, D_OUT), jnp.float32),
    )(agg, degs, b1, w2)


def _final(agg2, degs, b2):
    return pl.pallas_call(
        _final_body,
        grid=(NPAD // R_TC,),
        in_specs=[
            pl.BlockSpec((NC, R_TC, D_OUT), lambda i: (0, i, 0)),
            _DEG_SPEC,
            pl.BlockSpec((1, D_OUT), lambda i: (0, 0)),
        ],
        out_specs=pl.BlockSpec((R_TC, D_OUT), lambda i: (i, 0)),
        out_shape=jax.ShapeDtypeStruct((NPAD, D_OUT), jnp.float32),
    )(agg2, degs, b2)


def kernel(features, edge_index, W1, b1, W2, b2):
    src = edge_index[0].astype(jnp.int32)
    dst = edge_index[1].astype(jnp.int32)
    pad = ((0, 0), (0, EPT_PAD - EPT))
    src_p = jnp.pad(src.reshape(NW, EPT), pad, constant_values=N)
    dst_p = jnp.pad(dst.reshape(NW, EPT), pad, constant_values=N)
    src_p = src_p.reshape(NW, NCH, CH)
    dst_p = dst_p.reshape(NW, NCH, CH)

    x_pad = jnp.pad(features, ((0, NPAD - N), (0, 0)))
    ones8 = jnp.ones((CH, 8), jnp.float32)
    z8 = jnp.zeros((RPT, 8), jnp.float32)
    z128 = jnp.zeros((RPT, D_HID), jnp.float32)
    z16 = jnp.zeros((RPT, D_OUT), jnp.float32)

    src_p64 = src_p.reshape(NW, EPT_PAD // CH128, CH128)
    dst_p64 = dst_p.reshape(NW, EPT_PAD // CH128, CH128)

    degs = _degrees(src_p, dst_p, ones8, z8)
    h1 = _mm1(x_pad, W1, degs)
    agg1 = _segsum128(h1, src_p64, dst_p64, z128)
    h2 = _mm2(agg1, degs, b1.reshape(1, D_HID), W2)
    agg2 = _segsum16(h2, src_p, dst_p, z16)
    out = _final(agg2, degs, b2.reshape(1, D_OUT))
    return out[:N]


# speedup vs baseline: 1.0000x; 1.0000x over previous
import functools

import jax
import jax.numpy as jnp
from jax import lax
from jax.experimental import pallas as pl
from jax.experimental.pallas import tpu as pltpu
from jax.experimental.pallas import tpu_sc as plsc

N = 10000
NPAD = 10112
E = 320000
D_IN = 128
D_HID = 128
D_OUT = 16
NC, NS = 2, 16
NW = NC * NS
CH = 128
EPT = E // NW
NCH = 80
EPT_PAD = NCH * CH
RPT = NPAD // NS
R_TC = 2528

_mesh = plsc.VectorSubcoreMesh(
    core_axis_name="c", subcore_axis_name="s", num_cores=NC, num_subcores=NS)


@functools.partial(
    pl.kernel,
    out_type=jax.ShapeDtypeStruct((NC, 2, NPAD, 8), jnp.float32),
    mesh=_mesh,
    compiler_params=pltpu.CompilerParams(use_tc_tiling_on_sc=False),
    scratch_types=[
        pltpu.VMEM((NCH, CH), jnp.int32),
        pltpu.VMEM((NCH, CH), jnp.int32),
        pltpu.VMEM((CH, 8), jnp.float32),
        pltpu.VMEM_SHARED((NPAD, 8), jnp.float32),
        pltpu.VMEM_SHARED((NPAD, 8), jnp.float32),
    ],
)
def _degrees(src_hbm, dst_hbm, ones_hbm, zeros_hbm, out_hbm,
             src_v, dst_v, ones_v, acc_s, acc_d):
    cid = lax.axis_index("c")
    sid = lax.axis_index("s")
    wid = cid * NS + sid
    pltpu.sync_copy(zeros_hbm, acc_s.at[pl.ds(sid * RPT, RPT)])
    pltpu.sync_copy(zeros_hbm, acc_d.at[pl.ds(sid * RPT, RPT)])
    pltpu.sync_copy(ones_hbm, ones_v)
    pltpu.sync_copy(src_hbm.at[wid], src_v)
    pltpu.sync_copy(dst_hbm.at[wid], dst_v)
    plsc.subcore_barrier()

    def body(j, carry):
        pltpu.sync_copy(ones_v, acc_s.at[src_v.at[j]], add=True)
        pltpu.sync_copy(ones_v, acc_d.at[dst_v.at[j]], add=True)
        return carry

    lax.fori_loop(0, NCH, body, 0)
    plsc.subcore_barrier()
    rows = pl.ds(sid * RPT, RPT)
    pltpu.sync_copy(acc_s.at[rows], out_hbm.at[cid, 0, rows])
    pltpu.sync_copy(acc_d.at[rows], out_hbm.at[cid, 1, rows])


def _make_segsum(D, ch, qi, pipelined):
    nch = EPT_PAD // ch
    nq = nch // qi

    @functools.partial(
        pl.kernel,
        out_type=jax.ShapeDtypeStruct((NC, NPAD, D), jnp.float32),
        mesh=_mesh,
        compiler_params=pltpu.CompilerParams(use_tc_tiling_on_sc=(D == 128)),
        scratch_types=(
            [pltpu.VMEM((nq, ch), jnp.int32),
             pltpu.VMEM((nq, ch), jnp.int32)]
            + [pltpu.VMEM((ch, D), jnp.float32)] * (2 if pipelined else 1)
            + [pltpu.VMEM_SHARED((NPAD, D), jnp.float32)]
            + [pltpu.SemaphoreType.DMA] * (2 if pipelined else 1)
        ),
    )
    def segsum(table_hbm, src_hbm, dst_hbm, zeros_hbm, out_hbm,
               src_v, dst_v, *rest):
        if pipelined:
            rows0, rows1, acc, sem0, sem1 = rest
        else:
            rows0, acc, sem0 = rest
        cid = lax.axis_index("c")
        sid = lax.axis_index("s")
        wid = cid * NS + sid
        pltpu.sync_copy(zeros_hbm, acc.at[pl.ds(sid * RPT, RPT)])
        plsc.subcore_barrier()

        for q in range(qi):
            pltpu.sync_copy(src_hbm.at[wid, pl.ds(q * nq, nq)], src_v)
            pltpu.sync_copy(dst_hbm.at[wid, pl.ds(q * nq, nq)], dst_v)
            if pipelined:
                pltpu.async_copy(table_hbm.at[src_v.at[0]], rows0, sem0)
                pltpu.async_copy(table_hbm.at[src_v.at[1]], rows1, sem1)

                @pl.loop(0, nq - 2, step=2)
                def _(j):
                    pltpu.make_async_copy(table_hbm.at[src_v.at[j]], rows0, sem0).wait()
                    pltpu.sync_copy(rows0, acc.at[dst_v.at[j]], add=True)
                    pltpu.async_copy(table_hbm.at[src_v.at[j + 2]], rows0, sem0)
                    pltpu.make_async_copy(table_hbm.at[src_v.at[j + 1]], rows1, sem1).wait()
                    pltpu.sync_copy(rows1, acc.at[dst_v.at[j + 1]], add=True)
                    pltpu.async_copy(table_hbm.at[src_v.at[j + 3]], rows1, sem1)

                pltpu.make_async_copy(table_hbm.at[src_v.at[nq - 2]], rows0, sem0).wait()
                pltpu.sync_copy(rows0, acc.at[dst_v.at[nq - 2]], add=True)
                pltpu.make_async_copy(table_hbm.at[src_v.at[nq - 1]], rows1, sem1).wait()
                pltpu.sync_copy(rows1, acc.at[dst_v.at[nq - 1]], add=True)
            else:
                def body(j, carry):
                    pltpu.async_copy(table_hbm.at[src_v.at[j]], rows0, sem0).wait()
                    pltpu.sync_copy(rows0, acc.at[dst_v.at[j]], add=True)
                    return carry

                lax.fori_loop(0, nq, body, 0)
        plsc.subcore_barrier()
        rows = pl.ds(sid * RPT, RPT)
        pltpu.sync_copy(acc.at[rows], out_hbm.at[cid, rows])

    return segsum


CH128 = 128
_segsum128 = _make_segsum(D_HID, CH128, 1, pipelined=False)
_segsum16 = _make_segsum(D_OUT, CH, 1, pipelined=True)


def _norm_from(deg_ref, which):
    deg = deg_ref[0, which][:, :1] + deg_ref[1, which][:, :1]
    return lax.rsqrt(jnp.maximum(deg, 1.0))


def _mm1_body(x_ref, w_ref, deg_ref, o_ref):
    norm_out = _norm_from(deg_ref, 0)
    o_ref[...] = jnp.dot(x_ref[...], w_ref[...],
                         preferred_element_type=jnp.float32) * norm_out


def _mm2_body(agg_ref, deg_ref, b1_ref, w2_ref, o_ref):
    agg = agg_ref[0] + agg_ref[1]
    norm_in = _norm_from(deg_ref, 1)
    norm_out = _norm_from(deg_ref, 0)
    h = jnp.maximum(agg * norm_in + b1_ref[...], 0.0)
    h2 = jnp.dot(h, w2_ref[...], preferred_element_type=jnp.float32) * norm_out
    rows = lax.broadcasted_iota(jnp.int32, (R_TC, 1), 0) + pl.program_id(0) * R_TC
    o_ref[...] = jnp.where(rows < N, h2, 0.0)


def _final_body(agg_ref, deg_ref, b2_ref, o_ref):
    agg = agg_ref[0] + agg_ref[1]
    norm_in = _norm_from(deg_ref, 1)
    o_ref[...] = agg * norm_in + b2_ref[...]


_DEG_SPEC = pl.BlockSpec((NC, 2, R_TC, 8), lambda i: (0, 0, i, 0))


def _mm1(x, w1, degs):
    return pl.pallas_call(
        _mm1_body,
        grid=(NPAD // R_TC,),
        in_specs=[
            pl.BlockSpec((R_TC, D_IN), lambda i: (i, 0)),
            pl.BlockSpec((D_IN, D_HID), lambda i: (0, 0)),
            _DEG_SPEC,
        ],
        out_specs=pl.BlockSpec((R_TC, D_HID), lambda i: (i, 0)),
        out_shape=jax.ShapeDtypeStruct((NPAD, D_HID), jnp.float32),
    )(x, w1, degs)


def _mm2(agg, degs, b1, w2):
    return pl.pallas_call(
        _mm2_body,
        grid=(NPAD // R_TC,),
        in_specs=[
            pl.BlockSpec((NC, R_TC, D_HID), lambda i: (0, i, 0)),
            _DEG_SPEC,
            pl.BlockSpec((1, D_HID), lambda i: (0, 0)),
            pl.BlockSpec((D_HID, D_OUT), lambda i: (0, 0)),
        ],
        out_specs=pl.BlockSpec((R_TC, D_OUT), lambda i: (i, 0)),
        out_shape=jax.ShapeDtypeStruct((NPAD, D_OUT), jnp.float32),
    )(agg, degs, b1, w2)


def _final(agg2, degs, b2):
    return pl.pallas_call(
        _final_body,
        grid=(NPAD // R_TC,),
        in_specs=[
            pl.BlockSpec((NC, R_TC, D_OUT), lambda i: (0, i, 0)),
            _DEG_SPEC,
            pl.BlockSpec((1, D_OUT), lambda i: (0, 0)),
        ],
        out_specs=pl.BlockSpec((R_TC, D_OUT), lambda i: (i, 0)),
        out_shape=jax.ShapeDtypeStruct((NPAD, D_OUT), jnp.float32),
    )(agg2, degs, b2)


def kernel(features, edge_index, W1, b1, W2, b2):
    src = edge_index[0].astype(jnp.int32)
    dst = edge_index[1].astype(jnp.int32)
    pad = ((0, 0), (0, EPT_PAD - EPT))
    src_p = jnp.pad(src.reshape(NW, EPT), pad, constant_values=N)
    dst_p = jnp.pad(dst.reshape(NW, EPT), pad, constant_values=N)
    src_p = src_p.reshape(NW, NCH, CH)
    dst_p = dst_p.reshape(NW, NCH, CH)

    x_pad = jnp.pad(features, ((0, NPAD - N), (0, 0)))
    ones8 = jnp.ones((CH, 8), jnp.float32)
    z8 = jnp.zeros((RPT, 8), jnp.float32)
    z128 = jnp.zeros((RPT, D_HID), jnp.float32)
    z16 = jnp.zeros((RPT, D_OUT), jnp.float32)

    src_p64 = src_p.reshape(NW, EPT_PAD // CH128, CH128)
    dst_p64 = dst_p.reshape(NW, EPT_PAD // CH128, CH128)

    degs = _degrees(src_p, dst_p, ones8, z8)
    h1 = _mm1(x_pad, W1, degs)
    agg1 = _segsum128(h1, src_p64, dst_p64, z128)
    h2 = _mm2(agg1, degs, b1.reshape(1, D_HID), W2)
    agg2 = _segsum16(h2, src_p, dst_p, z16)
    out = _final(agg2, degs, b2.reshape(1, D_OUT))
    return out[:N]


# speedup vs baseline: 1.8862x; 1.8862x over previous
import functools

import numpy as np
import jax
import jax.numpy as jnp
from jax import lax
from jax.experimental import pallas as pl
from jax.experimental.pallas import tpu as pltpu
from jax.experimental.pallas import tpu_sc as plsc

N = 10000
NPAD = 10240
E = 320000
D_IN = 128
D_HID = 128
D_OUT = 16
NC, NS = 2, 16
NW = NC * NS
CH = 128
EPT = E // NW
NCH = 80
EPT_PAD = NCH * CH
PADE = EPT_PAD - EPT
RPT = NPAD // NS
R_TC = 2560

_PAD_IDX = jnp.asarray(
    N + (np.arange(PADE)[None, :] + 16 * np.arange(NW)[:, None]) % (NPAD - N),
    dtype=jnp.int32)

_mesh = plsc.VectorSubcoreMesh(
    core_axis_name="c", subcore_axis_name="s", num_cores=NC, num_subcores=NS)


@functools.partial(
    pl.kernel,
    out_type=jax.ShapeDtypeStruct((NC, 2, NPAD, 8), jnp.float32),
    mesh=_mesh,
    compiler_params=pltpu.CompilerParams(use_tc_tiling_on_sc=False),
    scratch_types=[
        pltpu.VMEM((NCH, CH), jnp.int32),
        pltpu.VMEM((NCH, CH), jnp.int32),
        pltpu.VMEM((CH, 8), jnp.float32),
        pltpu.VMEM_SHARED((NPAD, 8), jnp.float32),
        pltpu.VMEM_SHARED((NPAD, 8), jnp.float32),
    ],
)
def _degrees(src_hbm, dst_hbm, ones_hbm, zeros_hbm, out_hbm,
             src_v, dst_v, ones_v, acc_s, acc_d):
    cid = lax.axis_index("c")
    sid = lax.axis_index("s")
    wid = cid * NS + sid
    pltpu.sync_copy(zeros_hbm, acc_s.at[pl.ds(sid * RPT, RPT)])
    pltpu.sync_copy(zeros_hbm, acc_d.at[pl.ds(sid * RPT, RPT)])
    pltpu.sync_copy(ones_hbm, ones_v)
    pltpu.sync_copy(src_hbm.at[wid], src_v)
    pltpu.sync_copy(dst_hbm.at[wid], dst_v)
    plsc.subcore_barrier()

    def body(j, carry):
        pltpu.sync_copy(ones_v, acc_s.at[src_v.at[j]], add=True)
        pltpu.sync_copy(ones_v, acc_d.at[dst_v.at[j]], add=True)
        return carry

    lax.fori_loop(0, NCH, body, 0)
    plsc.subcore_barrier()
    rows = pl.ds(sid * RPT, RPT)
    pltpu.sync_copy(acc_s.at[rows], out_hbm.at[cid, 0, rows])
    pltpu.sync_copy(acc_d.at[rows], out_hbm.at[cid, 1, rows])


def _make_segsum(D, ch, qi, pipelined):
    nch = EPT_PAD // ch
    nq = nch // qi

    @functools.partial(
        pl.kernel,
        out_type=jax.ShapeDtypeStruct((NC, NPAD, D), jnp.float32),
        mesh=_mesh,
        compiler_params=pltpu.CompilerParams(use_tc_tiling_on_sc=(D == 128)),
        scratch_types=(
            [pltpu.VMEM((nq, ch), jnp.int32),
             pltpu.VMEM((nq, ch), jnp.int32)]
            + [pltpu.VMEM((ch, D), jnp.float32)] * (2 if pipelined else 1)
            + [pltpu.VMEM_SHARED((NPAD, D), jnp.float32)]
            + [pltpu.SemaphoreType.DMA] * (2 if pipelined else 1)
        ),
    )
    def segsum(table_hbm, src_hbm, dst_hbm, zeros_hbm, out_hbm,
               src_v, dst_v, *rest):
        if pipelined:
            rows0, rows1, acc, sem0, sem1 = rest
        else:
            rows0, acc, sem0 = rest
        cid = lax.axis_index("c")
        sid = lax.axis_index("s")
        wid = cid * NS + sid
        pltpu.sync_copy(zeros_hbm, acc.at[pl.ds(sid * RPT, RPT)])
        plsc.subcore_barrier()

        for q in range(qi):
            pltpu.sync_copy(src_hbm.at[wid, pl.ds(q * nq, nq)], src_v)
            pltpu.sync_copy(dst_hbm.at[wid, pl.ds(q * nq, nq)], dst_v)
            if pipelined:
                pltpu.async_copy(table_hbm.at[src_v.at[0]], rows0, sem0)
                pltpu.async_copy(table_hbm.at[src_v.at[1]], rows1, sem1)

                @pl.loop(0, nq - 2, step=2)
                def _(j):
                    pltpu.make_async_copy(table_hbm.at[src_v.at[j]], rows0, sem0).wait()
                    pltpu.sync_copy(rows0, acc.at[dst_v.at[j]], add=True)
                    pltpu.async_copy(table_hbm.at[src_v.at[j + 2]], rows0, sem0)
                    pltpu.make_async_copy(table_hbm.at[src_v.at[j + 1]], rows1, sem1).wait()
                    pltpu.sync_copy(rows1, acc.at[dst_v.at[j + 1]], add=True)
                    pltpu.async_copy(table_hbm.at[src_v.at[j + 3]], rows1, sem1)

                pltpu.make_async_copy(table_hbm.at[src_v.at[nq - 2]], rows0, sem0).wait()
                pltpu.sync_copy(rows0, acc.at[dst_v.at[nq - 2]], add=True)
                pltpu.make_async_copy(table_hbm.at[src_v.at[nq - 1]], rows1, sem1).wait()
                pltpu.sync_copy(rows1, acc.at[dst_v.at[nq - 1]], add=True)
            else:
                def body(j, carry):
                    pltpu.async_copy(table_hbm.at[src_v.at[j]], rows0, sem0).wait()
                    pltpu.sync_copy(rows0, acc.at[dst_v.at[j]], add=True)
                    return carry

                lax.fori_loop(0, nq, body, 0)
        plsc.subcore_barrier()
        rows = pl.ds(sid * RPT, RPT)
        pltpu.sync_copy(acc.at[rows], out_hbm.at[cid, rows])

    return segsum


CH128 = 128
_segsum128 = _make_segsum(D_HID, CH128, 1, pipelined=False)
_segsum16 = _make_segsum(D_OUT, CH, 1, pipelined=True)


def _norm_from(deg_ref, which):
    deg = deg_ref[0, which][:, :1] + deg_ref[1, which][:, :1]
    return lax.rsqrt(jnp.maximum(deg, 1.0))


def _mm1_body(x_ref, w_ref, deg_ref, o_ref):
    norm_out = _norm_from(deg_ref, 0)
    o_ref[...] = jnp.dot(x_ref[...], w_ref[...],
                         preferred_element_type=jnp.float32) * norm_out


def _mm2_body(agg_ref, deg_ref, b1_ref, w2_ref, o_ref):
    agg = agg_ref[0] + agg_ref[1]
    norm_in = _norm_from(deg_ref, 1)
    norm_out = _norm_from(deg_ref, 0)
    h = jnp.maximum(agg * norm_in + b1_ref[...], 0.0)
    h2 = jnp.dot(h, w2_ref[...], preferred_element_type=jnp.float32) * norm_out
    rows = lax.broadcasted_iota(jnp.int32, (R_TC, 1), 0) + pl.program_id(0) * R_TC
    o_ref[...] = jnp.where(rows < N, h2, 0.0)


def _final_body(agg_ref, deg_ref, b2_ref, o_ref):
    agg = agg_ref[0] + agg_ref[1]
    norm_in = _norm_from(deg_ref, 1)
    o_ref[...] = agg * norm_in + b2_ref[...]


_DEG_SPEC = pl.BlockSpec((NC, 2, R_TC, 8), lambda i: (0, 0, i, 0))


def _mm1(x, w1, degs):
    return pl.pallas_call(
        _mm1_body,
        grid=(NPAD // R_TC,),
        in_specs=[
            pl.BlockSpec((R_TC, D_IN), lambda i: (i, 0)),
            pl.BlockSpec((D_IN, D_HID), lambda i: (0, 0)),
            _DEG_SPEC,
        ],
        out_specs=pl.BlockSpec((R_TC, D_HID), lambda i: (i, 0)),
        out_shape=jax.ShapeDtypeStruct((NPAD, D_HID), jnp.float32),
    )(x, w1, degs)


def _mm2(agg, degs, b1, w2):
    return pl.pallas_call(
        _mm2_body,
        grid=(NPAD // R_TC,),
        in_specs=[
            pl.BlockSpec((NC, R_TC, D_HID), lambda i: (0, i, 0)),
            _DEG_SPEC,
            pl.BlockSpec((1, D_HID), lambda i: (0, 0)),
            pl.BlockSpec((D_HID, D_OUT), lambda i: (0, 0)),
        ],
        out_specs=pl.BlockSpec((R_TC, D_OUT), lambda i: (i, 0)),
        out_shape=jax.ShapeDtypeStruct((NPAD, D_OUT), jnp.float32),
    )(agg, degs, b1, w2)


def _final(agg2, degs, b2):
    return pl.pallas_call(
        _final_body,
        grid=(NPAD // R_TC,),
        in_specs=[
            pl.BlockSpec((NC, R_TC, D_OUT), lambda i: (0, i, 0)),
            _DEG_SPEC,
            pl.BlockSpec((1, D_OUT), lambda i: (0, 0)),
        ],
        out_specs=pl.BlockSpec((R_TC, D_OUT), lambda i: (i, 0)),
        out_shape=jax.ShapeDtypeStruct((NPAD, D_OUT), jnp.float32),
    )(agg2, degs, b2)


def kernel(features, edge_index, W1, b1, W2, b2):
    src = edge_index[0].astype(jnp.int32)
    dst = edge_index[1].astype(jnp.int32)
    src_p = jnp.concatenate([src.reshape(NW, EPT), _PAD_IDX], axis=1)
    dst_p = jnp.concatenate([dst.reshape(NW, EPT), _PAD_IDX], axis=1)
    src_p = src_p.reshape(NW, NCH, CH)
    dst_p = dst_p.reshape(NW, NCH, CH)

    x_pad = jnp.pad(features, ((0, NPAD - N), (0, 0)))
    ones8 = jnp.ones((CH, 8), jnp.float32)
    z8 = jnp.zeros((RPT, 8), jnp.float32)
    z128 = jnp.zeros((RPT, D_HID), jnp.float32)
    z16 = jnp.zeros((RPT, D_OUT), jnp.float32)

    src_p64 = src_p.reshape(NW, EPT_PAD // CH128, CH128)
    dst_p64 = dst_p.reshape(NW, EPT_PAD // CH128, CH128)

    degs = _degrees(src_p, dst_p, ones8, z8)
    h1 = _mm1(x_pad, W1, degs)
    agg1 = _segsum128(h1, src_p64, dst_p64, z128)
    h2 = _mm2(agg1, degs, b1.reshape(1, D_HID), W2)
    agg2 = _segsum16(h2, src_p, dst_p, z16)
    out = _final(agg2, degs, b2.reshape(1, D_OUT))
    return out[:N]


# speedup vs baseline: 2.3151x; 1.2274x over previous
import functools

import numpy as np
import jax
import jax.numpy as jnp
from jax import lax
from jax.experimental import pallas as pl
from jax.experimental.pallas import tpu as pltpu
from jax.experimental.pallas import tpu_sc as plsc

N = 10000
NPAD = 10240
E = 320000
D_IN = 128
D_HID = 128
D_OUT = 16
NC, NS = 2, 16
NW = NC * NS
CH = 128
EPT = E // NW
NCH = 80
EPT_PAD = NCH * CH
PADE = EPT_PAD - EPT
RPT = NPAD // NS
R_TC = 2560

_PAD_IDX = jnp.asarray(
    N + (np.arange(PADE)[None, :] + 16 * np.arange(NW)[:, None]) % (NPAD - N),
    dtype=jnp.int32)

_mesh = plsc.VectorSubcoreMesh(
    core_axis_name="c", subcore_axis_name="s", num_cores=NC, num_subcores=NS)


@functools.partial(
    pl.kernel,
    out_type=jax.ShapeDtypeStruct((NC, 2, NPAD, 8), jnp.float32),
    mesh=_mesh,
    compiler_params=pltpu.CompilerParams(use_tc_tiling_on_sc=False),
    scratch_types=[
        pltpu.VMEM((NCH, CH), jnp.int32),
        pltpu.VMEM((NCH, CH), jnp.int32),
        pltpu.VMEM((CH, 8), jnp.float32),
        pltpu.VMEM_SHARED((NPAD, 8), jnp.float32),
        pltpu.VMEM_SHARED((NPAD, 8), jnp.float32),
    ],
)
def _degrees(src_hbm, dst_hbm, ones_hbm, zeros_hbm, out_hbm,
             src_v, dst_v, ones_v, acc_s, acc_d):
    cid = lax.axis_index("c")
    sid = lax.axis_index("s")
    wid = cid * NS + sid
    pltpu.sync_copy(zeros_hbm, acc_s.at[pl.ds(sid * RPT, RPT)])
    pltpu.sync_copy(zeros_hbm, acc_d.at[pl.ds(sid * RPT, RPT)])
    pltpu.sync_copy(ones_hbm, ones_v)
    pltpu.sync_copy(src_hbm.at[wid], src_v)
    pltpu.sync_copy(dst_hbm.at[wid], dst_v)
    plsc.subcore_barrier()

    def body(j, carry):
        pltpu.sync_copy(ones_v, acc_s.at[src_v.at[j]], add=True)
        pltpu.sync_copy(ones_v, acc_d.at[dst_v.at[j]], add=True)
        return carry

    lax.fori_loop(0, NCH, body, 0)
    plsc.subcore_barrier()
    rows = pl.ds(sid * RPT, RPT)
    pltpu.sync_copy(acc_s.at[rows], out_hbm.at[cid, 0, rows])
    pltpu.sync_copy(acc_d.at[rows], out_hbm.at[cid, 1, rows])


def _make_segsum(D, ch, qi, pipelined):
    nch = EPT_PAD // ch
    nq = nch // qi

    @functools.partial(
        pl.kernel,
        out_type=jax.ShapeDtypeStruct((NC, NPAD, D), jnp.float32),
        mesh=_mesh,
        compiler_params=pltpu.CompilerParams(use_tc_tiling_on_sc=(D == 128)),
        scratch_types=(
            [pltpu.VMEM((nq, ch), jnp.int32),
             pltpu.VMEM((nq, ch), jnp.int32)]
            + [pltpu.VMEM((ch, D), jnp.float32)] * (2 if pipelined else 1)
            + [pltpu.VMEM_SHARED((NPAD, D), jnp.float32)]
            + [pltpu.SemaphoreType.DMA] * (2 if pipelined else 1)
        ),
    )
    def segsum(table_hbm, src_hbm, dst_hbm, zeros_hbm, out_hbm,
               src_v, dst_v, *rest):
        if pipelined:
            rows0, rows1, acc, sem0, sem1 = rest
        else:
            rows0, acc, sem0 = rest
        cid = lax.axis_index("c")
        sid = lax.axis_index("s")
        wid = cid * NS + sid
        pltpu.sync_copy(zeros_hbm, acc.at[pl.ds(sid * RPT, RPT)])
        plsc.subcore_barrier()

        for q in range(qi):
            pltpu.sync_copy(src_hbm.at[wid, pl.ds(q * nq, nq)], src_v)
            pltpu.sync_copy(dst_hbm.at[wid, pl.ds(q * nq, nq)], dst_v)
            if pipelined:
                pltpu.async_copy(table_hbm.at[src_v.at[0]], rows0, sem0)
                pltpu.async_copy(table_hbm.at[src_v.at[1]], rows1, sem1)

                @pl.loop(0, nq - 2, step=2)
                def _(j):
                    pltpu.make_async_copy(table_hbm.at[src_v.at[j]], rows0, sem0).wait()
                    pltpu.sync_copy(rows0, acc.at[dst_v.at[j]], add=True)
                    pltpu.async_copy(table_hbm.at[src_v.at[j + 2]], rows0, sem0)
                    pltpu.make_async_copy(table_hbm.at[src_v.at[j + 1]], rows1, sem1).wait()
                    pltpu.sync_copy(rows1, acc.at[dst_v.at[j + 1]], add=True)
                    pltpu.async_copy(table_hbm.at[src_v.at[j + 3]], rows1, sem1)

                pltpu.make_async_copy(table_hbm.at[src_v.at[nq - 2]], rows0, sem0).wait()
                pltpu.sync_copy(rows0, acc.at[dst_v.at[nq - 2]], add=True)
                pltpu.make_async_copy(table_hbm.at[src_v.at[nq - 1]], rows1, sem1).wait()
                pltpu.sync_copy(rows1, acc.at[dst_v.at[nq - 1]], add=True)
            else:
                def body(j, carry):
                    pltpu.async_copy(table_hbm.at[src_v.at[j]], rows0, sem0).wait()
                    pltpu.sync_copy(rows0, acc.at[dst_v.at[j]], add=True)
                    return carry

                lax.fori_loop(0, nq, body, 0)
        plsc.subcore_barrier()
        rows = pl.ds(sid * RPT, RPT)
        pltpu.sync_copy(acc.at[rows], out_hbm.at[cid, rows])

    return segsum


CH128 = 128
_segsum128 = _make_segsum(D_HID, CH128, 2, pipelined=True)
_segsum16 = _make_segsum(D_OUT, CH, 1, pipelined=True)


def _norm_from(deg_ref, which):
    deg = deg_ref[0, which][:, :1] + deg_ref[1, which][:, :1]
    return lax.rsqrt(jnp.maximum(deg, 1.0))


def _mm1_body(x_ref, w_ref, deg_ref, o_ref):
    norm_out = _norm_from(deg_ref, 0)
    o_ref[...] = jnp.dot(x_ref[...], w_ref[...],
                         preferred_element_type=jnp.float32) * norm_out


def _mm2_body(agg_ref, deg_ref, b1_ref, w2_ref, o_ref):
    agg = agg_ref[0] + agg_ref[1]
    norm_in = _norm_from(deg_ref, 1)
    norm_out = _norm_from(deg_ref, 0)
    h = jnp.maximum(agg * norm_in + b1_ref[...], 0.0)
    h2 = jnp.dot(h, w2_ref[...], preferred_element_type=jnp.float32) * norm_out
    rows = lax.broadcasted_iota(jnp.int32, (R_TC, 1), 0) + pl.program_id(0) * R_TC
    o_ref[...] = jnp.where(rows < N, h2, 0.0)


def _final_body(agg_ref, deg_ref, b2_ref, o_ref):
    agg = agg_ref[0] + agg_ref[1]
    norm_in = _norm_from(deg_ref, 1)
    o_ref[...] = agg * norm_in + b2_ref[...]


_DEG_SPEC = pl.BlockSpec((NC, 2, R_TC, 8), lambda i: (0, 0, i, 0))


def _mm1(x, w1, degs):
    return pl.pallas_call(
        _mm1_body,
        grid=(NPAD // R_TC,),
        in_specs=[
            pl.BlockSpec((R_TC, D_IN), lambda i: (i, 0)),
            pl.BlockSpec((D_IN, D_HID), lambda i: (0, 0)),
            _DEG_SPEC,
        ],
        out_specs=pl.BlockSpec((R_TC, D_HID), lambda i: (i, 0)),
        out_shape=jax.ShapeDtypeStruct((NPAD, D_HID), jnp.float32),
    )(x, w1, degs)


def _mm2(agg, degs, b1, w2):
    return pl.pallas_call(
        _mm2_body,
        grid=(NPAD // R_TC,),
        in_specs=[
            pl.BlockSpec((NC, R_TC, D_HID), lambda i: (0, i, 0)),
            _DEG_SPEC,
            pl.BlockSpec((1, D_HID), lambda i: (0, 0)),
            pl.BlockSpec((D_HID, D_OUT), lambda i: (0, 0)),
        ],
        out_specs=pl.BlockSpec((R_TC, D_OUT), lambda i: (i, 0)),
        out_shape=jax.ShapeDtypeStruct((NPAD, D_OUT), jnp.float32),
    )(agg, degs, b1, w2)


def _final(agg2, degs, b2):
    return pl.pallas_call(
        _final_body,
        grid=(NPAD // R_TC,),
        in_specs=[
            pl.BlockSpec((NC, R_TC, D_OUT), lambda i: (0, i, 0)),
            _DEG_SPEC,
            pl.BlockSpec((1, D_OUT), lambda i: (0, 0)),
        ],
        out_specs=pl.BlockSpec((R_TC, D_OUT), lambda i: (i, 0)),
        out_shape=jax.ShapeDtypeStruct((NPAD, D_OUT), jnp.float32),
    )(agg2, degs, b2)


def kernel(features, edge_index, W1, b1, W2, b2):
    src = edge_index[0].astype(jnp.int32)
    dst = edge_index[1].astype(jnp.int32)
    src_p = jnp.concatenate([src.reshape(NW, EPT), _PAD_IDX], axis=1)
    dst_p = jnp.concatenate([dst.reshape(NW, EPT), _PAD_IDX], axis=1)
    src_p = src_p.reshape(NW, NCH, CH)
    dst_p = dst_p.reshape(NW, NCH, CH)

    x_pad = jnp.pad(features, ((0, NPAD - N), (0, 0)))
    ones8 = jnp.ones((CH, 8), jnp.float32)
    z8 = jnp.zeros((RPT, 8), jnp.float32)
    z128 = jnp.zeros((RPT, D_HID), jnp.float32)
    z16 = jnp.zeros((RPT, D_OUT), jnp.float32)

    src_p64 = src_p.reshape(NW, EPT_PAD // CH128, CH128)
    dst_p64 = dst_p.reshape(NW, EPT_PAD // CH128, CH128)

    degs = _degrees(src_p, dst_p, ones8, z8)
    h1 = _mm1(x_pad, W1, degs)
    agg1 = _segsum128(h1, src_p64, dst_p64, z128)
    h2 = _mm2(agg1, degs, b1.reshape(1, D_HID), W2)
    agg2 = _segsum16(h2, src_p, dst_p, z16)
    out = _final(agg2, degs, b2.reshape(1, D_OUT))
    return out[:N]


# speedup vs baseline: 2.5312x; 1.0933x over previous
import functools

import numpy as np
import jax
import jax.numpy as jnp
from jax import lax
from jax.experimental import pallas as pl
from jax.experimental.pallas import tpu as pltpu
from jax.experimental.pallas import tpu_sc as plsc

N = 10000
NPAD = 10240
E = 320000
D_IN = 128
D_HID = 128
D_OUT = 16
NC, NS = 2, 16
NW = NC * NS
CH = 128
EPT = E // NW
NCH = 80
EPT_PAD = NCH * CH
PADE = EPT_PAD - EPT
RPT = NPAD // NS
R_TC = 2560

_PAD_IDX = jnp.asarray(
    N + (np.arange(PADE)[None, :] + 16 * np.arange(NW)[:, None]) % (NPAD - N),
    dtype=jnp.int32)

_mesh = plsc.VectorSubcoreMesh(
    core_axis_name="c", subcore_axis_name="s", num_cores=NC, num_subcores=NS)


@functools.partial(
    pl.kernel,
    out_type=jax.ShapeDtypeStruct((NC, 2, NPAD, 8), jnp.float32),
    mesh=_mesh,
    compiler_params=pltpu.CompilerParams(use_tc_tiling_on_sc=False),
    scratch_types=[
        pltpu.VMEM((NCH, CH), jnp.int32),
        pltpu.VMEM((NCH, CH), jnp.int32),
        pltpu.VMEM((CH, 8), jnp.float32),
        pltpu.VMEM_SHARED((NPAD, 8), jnp.float32),
        pltpu.VMEM_SHARED((NPAD, 8), jnp.float32),
        pltpu.SemaphoreType.DMA,
        pltpu.SemaphoreType.DMA,
    ],
)
def _degrees(src_hbm, dst_hbm, ones_hbm, zeros_hbm, out_hbm,
             src_v, dst_v, ones_v, acc_s, acc_d, sem_a, sem_b):
    cid = lax.axis_index("c")
    sid = lax.axis_index("s")
    wid = cid * NS + sid
    pltpu.sync_copy(zeros_hbm, acc_s.at[pl.ds(sid * RPT, RPT)])
    pltpu.sync_copy(zeros_hbm, acc_d.at[pl.ds(sid * RPT, RPT)])
    pltpu.sync_copy(ones_hbm, ones_v)
    pltpu.sync_copy(src_hbm.at[wid], src_v)
    pltpu.sync_copy(dst_hbm.at[wid], dst_v)
    plsc.subcore_barrier()

    pltpu.async_copy(ones_v, acc_s.at[src_v.at[0]], sem_a, add=True)
    pltpu.async_copy(ones_v, acc_d.at[dst_v.at[0]], sem_b, add=True)

    def body(j, carry):
        pltpu.async_copy(ones_v, acc_s.at[src_v.at[j]], sem_a, add=True)
        pltpu.async_copy(ones_v, acc_d.at[dst_v.at[j]], sem_b, add=True)
        pltpu.make_async_copy(ones_v, acc_s.at[src_v.at[j]], sem_a).wait()
        pltpu.make_async_copy(ones_v, acc_d.at[dst_v.at[j]], sem_b).wait()
        return carry

    lax.fori_loop(1, NCH, body, 0)
    pltpu.make_async_copy(ones_v, acc_s.at[src_v.at[0]], sem_a).wait()
    pltpu.make_async_copy(ones_v, acc_d.at[dst_v.at[0]], sem_b).wait()
    plsc.subcore_barrier()
    rows = pl.ds(sid * RPT, RPT)
    pltpu.sync_copy(acc_s.at[rows], out_hbm.at[cid, 0, rows])
    pltpu.sync_copy(acc_d.at[rows], out_hbm.at[cid, 1, rows])


def _make_segsum(D, ch, qi, pipelined):
    nch = EPT_PAD // ch
    nq = nch // qi
    nbuf = pipelined if pipelined else 1

    @functools.partial(
        pl.kernel,
        out_type=jax.ShapeDtypeStruct((NC, NPAD, D), jnp.float32),
        mesh=_mesh,
        compiler_params=pltpu.CompilerParams(use_tc_tiling_on_sc=(D == 128)),
        scratch_types=(
            [pltpu.VMEM((nq, ch), jnp.int32),
             pltpu.VMEM((nq, ch), jnp.int32)]
            + [pltpu.VMEM((ch, D), jnp.float32)] * nbuf
            + [pltpu.VMEM_SHARED((NPAD, D), jnp.float32)]
            + [pltpu.SemaphoreType.DMA] * nbuf
        ),
    )
    def segsum(table_hbm, src_hbm, dst_hbm, zeros_hbm, out_hbm,
               src_v, dst_v, *rest):
        rows = rest[:nbuf]
        acc = rest[nbuf]
        sems = rest[nbuf + 1:]
        cid = lax.axis_index("c")
        sid = lax.axis_index("s")
        wid = cid * NS + sid
        pltpu.sync_copy(zeros_hbm, acc.at[pl.ds(sid * RPT, RPT)])
        plsc.subcore_barrier()

        for q in range(qi):
            pltpu.sync_copy(src_hbm.at[wid, pl.ds(q * nq, nq)], src_v)
            pltpu.sync_copy(dst_hbm.at[wid, pl.ds(q * nq, nq)], dst_v)
            if pipelined:
                for b in range(nbuf):
                    pltpu.async_copy(table_hbm.at[src_v.at[b]], rows[b], sems[b])

                @pl.loop(0, nq - nbuf, step=nbuf)
                def _(j):
                    for b in range(nbuf):
                        pltpu.make_async_copy(
                            table_hbm.at[src_v.at[j + b]], rows[b], sems[b]).wait()
                        pltpu.sync_copy(rows[b], acc.at[dst_v.at[j + b]], add=True)
                        pltpu.async_copy(
                            table_hbm.at[src_v.at[j + b + nbuf]], rows[b], sems[b])

                for b in range(nbuf):
                    jt = nq - nbuf + b
                    pltpu.make_async_copy(
                        table_hbm.at[src_v.at[jt]], rows[b], sems[b]).wait()
                    pltpu.sync_copy(rows[b], acc.at[dst_v.at[jt]], add=True)
            else:
                def body(j, carry):
                    pltpu.async_copy(table_hbm.at[src_v.at[j]], rows[0], sems[0]).wait()
                    pltpu.sync_copy(rows[0], acc.at[dst_v.at[j]], add=True)
                    return carry

                lax.fori_loop(0, nq, body, 0)
        plsc.subcore_barrier()
        wb = pl.ds(sid * RPT, RPT)
        pltpu.sync_copy(acc.at[wb], out_hbm.at[cid, wb])

    return segsum


CH128 = 128
_segsum128 = _make_segsum(D_HID, CH128, 2, pipelined=2)
_segsum16 = _make_segsum(D_OUT, CH, 1, pipelined=4)


def _norm_from(deg_ref, which):
    deg = deg_ref[0, which][:, :1] + deg_ref[1, which][:, :1]
    return lax.rsqrt(jnp.maximum(deg, 1.0))


def _mm1_body(x_ref, w_ref, deg_ref, o_ref):
    norm_out = _norm_from(deg_ref, 0)
    o_ref[...] = jnp.dot(x_ref[...], w_ref[...],
                         preferred_element_type=jnp.float32) * norm_out


def _mm2_body(agg_ref, deg_ref, b1_ref, w2_ref, o_ref):
    agg = agg_ref[0] + agg_ref[1]
    norm_in = _norm_from(deg_ref, 1)
    norm_out = _norm_from(deg_ref, 0)
    h = jnp.maximum(agg * norm_in + b1_ref[...], 0.0)
    h2 = jnp.dot(h, w2_ref[...], preferred_element_type=jnp.float32) * norm_out
    rows = lax.broadcasted_iota(jnp.int32, (R_TC, 1), 0) + pl.program_id(0) * R_TC
    o_ref[...] = jnp.where(rows < N, h2, 0.0)


def _final_body(agg_ref, deg_ref, b2_ref, o_ref):
    agg = agg_ref[0] + agg_ref[1]
    norm_in = _norm_from(deg_ref, 1)
    o_ref[...] = agg * norm_in + b2_ref[...]


_DEG_SPEC = pl.BlockSpec((NC, 2, R_TC, 8), lambda i: (0, 0, i, 0))


def _mm1(x, w1, degs):
    return pl.pallas_call(
        _mm1_body,
        grid=(NPAD // R_TC,),
        in_specs=[
            pl.BlockSpec((R_TC, D_IN), lambda i: (i, 0)),
            pl.BlockSpec((D_IN, D_HID), lambda i: (0, 0)),
            _DEG_SPEC,
        ],
        out_specs=pl.BlockSpec((R_TC, D_HID), lambda i: (i, 0)),
        out_shape=jax.ShapeDtypeStruct((NPAD, D_HID), jnp.float32),
    )(x, w1, degs)


def _mm2(agg, degs, b1, w2):
    return pl.pallas_call(
        _mm2_body,
        grid=(NPAD // R_TC,),
        in_specs=[
            pl.BlockSpec((NC, R_TC, D_HID), lambda i: (0, i, 0)),
            _DEG_SPEC,
            pl.BlockSpec((1, D_HID), lambda i: (0, 0)),
            pl.BlockSpec((D_HID, D_OUT), lambda i: (0, 0)),
        ],
        out_specs=pl.BlockSpec((R_TC, D_OUT), lambda i: (i, 0)),
        out_shape=jax.ShapeDtypeStruct((NPAD, D_OUT), jnp.float32),
    )(agg, degs, b1, w2)


def _final(agg2, degs, b2):
    return pl.pallas_call(
        _final_body,
        grid=(NPAD // R_TC,),
        in_specs=[
            pl.BlockSpec((NC, R_TC, D_OUT), lambda i: (0, i, 0)),
            _DEG_SPEC,
            pl.BlockSpec((1, D_OUT), lambda i: (0, 0)),
        ],
        out_specs=pl.BlockSpec((R_TC, D_OUT), lambda i: (i, 0)),
        out_shape=jax.ShapeDtypeStruct((NPAD, D_OUT), jnp.float32),
    )(agg2, degs, b2)


def kernel(features, edge_index, W1, b1, W2, b2):
    src = edge_index[0].astype(jnp.int32)
    dst = edge_index[1].astype(jnp.int32)
    src_p = jnp.concatenate([src.reshape(NW, EPT), _PAD_IDX], axis=1)
    dst_p = jnp.concatenate([dst.reshape(NW, EPT), _PAD_IDX], axis=1)
    src_p = src_p.reshape(NW, NCH, CH)
    dst_p = dst_p.reshape(NW, NCH, CH)

    x_pad = jnp.pad(features, ((0, NPAD - N), (0, 0)))
    ones8 = jnp.ones((CH, 8), jnp.float32)
    z8 = jnp.zeros((RPT, 8), jnp.float32)
    z128 = jnp.zeros((RPT, D_HID), jnp.float32)
    z16 = jnp.zeros((RPT, D_OUT), jnp.float32)

    src_p64 = src_p.reshape(NW, EPT_PAD // CH128, CH128)
    dst_p64 = dst_p.reshape(NW, EPT_PAD // CH128, CH128)

    degs = _degrees(src_p, dst_p, ones8, z8)
    h1 = _mm1(x_pad, W1, degs)
    agg1 = _segsum128(h1, src_p64, dst_p64, z128)
    h2 = _mm2(agg1, degs, b1.reshape(1, D_HID), W2)
    agg2 = _segsum16(h2, src_p, dst_p, z16)
    out = _final(agg2, degs, b2.reshape(1, D_OUT))
    return out[:N]
